# baseline (speedup 1.0000x reference)
import functools

import jax
import jax.numpy as jnp
from jax import lax
from jax.experimental import pallas as pl
from jax.experimental.pallas import tpu as pltpu
from jax.experimental.pallas import tpu_sc as plsc

N = 10000
E = 320000
F = 128
CH = 128
NG = 32
NOUT = 10

NCORES = 2
NSUB = 16
NTILES = NCORES * NSUB
CHUNK = 64
NCHUNK = 161
EPAD = NTILES * NCHUNK * CHUNK
NPAD = 10240
ROWS_PER_TILE = NPAD // NSUB
ZR = 16


def _sc_propagate(h, pk3, w3):
    mesh = plsc.VectorSubcoreMesh(core_axis_name="c", subcore_axis_name="s")

    @functools.partial(
        pl.kernel,
        out_type=jax.ShapeDtypeStruct((NCORES, NPAD, CH), jnp.float32),
        mesh=mesh,
        scratch_types=[
            pltpu.VMEM((NCHUNK, CHUNK), jnp.int32),
            pltpu.VMEM((CHUNK, CH), jnp.float32),
            pltpu.VMEM((CHUNK, CH), jnp.float32),
            pltpu.VMEM((CHUNK,), jnp.float32),
            pltpu.VMEM((CHUNK,), jnp.float32),
            pltpu.VMEM((CHUNK,), jnp.int32),
            pltpu.VMEM((CHUNK,), jnp.int32),
            pltpu.VMEM((CHUNK,), jnp.int32),
            pltpu.VMEM((ZR, CH), jnp.float32),
            pltpu.VMEM_SHARED((NPAD, CH), jnp.float32),
            pltpu.SemaphoreType.DMA,
            pltpu.SemaphoreType.DMA,
            pltpu.SemaphoreType.DMA,
            pltpu.SemaphoreType.DMA,
        ],
    )
    def prop(h_hbm, pk_hbm, w_hbm, out_hbm,
             mpack, rows_a, rows_b, w_a, w_b, sidx_a, sidx_b, didx,
             zbuf, acc, sem_a, sem_b, sem_wa, sem_wb):
        cid = lax.axis_index("c")
        sid = lax.axis_index("s")
        tile = cid * NSUB + sid

        pltpu.sync_copy(pk_hbm.at[tile], mpack)

        @pl.loop(0, ZR)
        def _zero(r):
            for j in range(CH // 16):
                zbuf[r, pl.ds(j * 16, 16)] = jnp.zeros((16,), jnp.float32)

        @pl.loop(0, ROWS_PER_TILE // ZR)
        def _zcopy(p_i):
            pltpu.sync_copy(
                zbuf, acc.at[pl.ds(sid * ROWS_PER_TILE + p_i * ZR, ZR)])
        plsc.subcore_barrier()

        def wait_rows(buf, sem, wbuf, sem_w):
            pltpu.make_async_copy(h_hbm.at[pl.ds(0, CHUNK)], buf, sem).wait()
            pltpu.make_async_copy(w_hbm.at[0, 0], wbuf, sem_w).wait()

        def scale_scatter(buf, wbuf, g):
            @pl.loop(0, CHUNK // 16)
            def _scale(gg):
                wvec = wbuf[pl.ds(gg * 16, 16)]
                for t in range(16):
                    e = gg * 16 + t
                    wv = jnp.full((16,), wvec[t], dtype=jnp.float32)
                    for j in range(CH // 16):
                        sl = pl.ds(j * 16, 16)
                        buf[e, sl] = buf[e, sl] * wv

            for gg in range(CHUNK // 16):
                sl = pl.ds(gg * 16, 16)
                didx[sl] = mpack[g, sl] & 16383
            pltpu.sync_copy(buf, acc.at[didx], add=True)

        def start_gather(buf, sem, wbuf, sem_w, sidx, g):
            for gg in range(CHUNK // 16):
                sl = pl.ds(gg * 16, 16)
                sidx[sl] = lax.shift_right_logical(mpack[g, sl], 14)
            pltpu.async_copy(h_hbm.at[sidx], buf, sem)
            pltpu.async_copy(w_hbm.at[tile, g], wbuf, sem_w)

        start_gather(rows_a, sem_a, w_a, sem_wa, sidx_a, 0)

        @pl.loop(0, (NCHUNK - 1) // 2)
        def _pair(p):
            g0 = 2 * p
            wait_rows(rows_a, sem_a, w_a, sem_wa)
            start_gather(rows_b, sem_b, w_b, sem_wb, sidx_b, g0 + 1)
            scale_scatter(rows_a, w_a, g0)
            wait_rows(rows_b, sem_b, w_b, sem_wb)
            start_gather(rows_a, sem_a, w_a, sem_wa, sidx_a, g0 + 2)
            scale_scatter(rows_b, w_b, g0 + 1)

        wait_rows(rows_a, sem_a, w_a, sem_wa)
        scale_scatter(rows_a, w_a, NCHUNK - 1)

        plsc.subcore_barrier()
        pltpu.sync_copy(
            acc.at[pl.ds(sid * ROWS_PER_TILE, ROWS_PER_TILE)],
            out_hbm.at[cid, pl.ds(sid * ROWS_PER_TILE, ROWS_PER_TILE)])

    return prop(h, pk3, w3)


RB = 1000
NRB = N // RB


def _elu(v):
    return jnp.where(v > 0, v, jnp.exp(v) - 1.0)


def _mm2_body(x_ref, k1_ref, k2_ref, h_ref, s_ref):
    xb = x_ref[...]
    h_ref[...] = jnp.dot(xb, k1_ref[...], preferred_element_type=jnp.float32)
    s_ref[...] = jnp.dot(xb, k2_ref[...], preferred_element_type=jnp.float32)


def _mm2(x, k1, k2):
    return pl.pallas_call(
        _mm2_body,
        grid=(NRB,),
        in_specs=[
            pl.BlockSpec((RB, F), lambda i: (i, 0)),
            pl.BlockSpec((F, CH), lambda i: (0, 0)),
            pl.BlockSpec((F, CH), lambda i: (0, 0)),
        ],
        out_specs=[
            pl.BlockSpec((RB, CH), lambda i: (i, 0)),
            pl.BlockSpec((RB, CH), lambda i: (i, 0)),
        ],
        out_shape=[
            jax.ShapeDtypeStruct((N, CH), jnp.float32),
            jax.ShapeDtypeStruct((N, CH), jnp.float32),
        ],
    )(x, k1, k2)


def _combine_mm2_body(p0_ref, p1_ref, s_ref, b_ref, k1_ref, k2_ref,
                      h_ref, s2_ref):
    out = _elu(_elu(p0_ref[...] + p1_ref[...] + s_ref[...] + b_ref[...]))
    h_ref[...] = jnp.dot(out, k1_ref[...], preferred_element_type=jnp.float32)
    s2_ref[...] = jnp.dot(out, k2_ref[...], preferred_element_type=jnp.float32)


def _combine_mm2(p0, p1, s, b, k1, k2):
    return pl.pallas_call(
        _combine_mm2_body,
        grid=(NRB,),
        in_specs=[
            pl.BlockSpec((RB, CH), lambda i: (i, 0)),
            pl.BlockSpec((RB, CH), lambda i: (i, 0)),
            pl.BlockSpec((RB, CH), lambda i: (i, 0)),
            pl.BlockSpec((1, CH), lambda i: (0, 0)),
            pl.BlockSpec((CH, CH), lambda i: (0, 0)),
            pl.BlockSpec((CH, CH), lambda i: (0, 0)),
        ],
        out_specs=[
            pl.BlockSpec((RB, CH), lambda i: (i, 0)),
            pl.BlockSpec((RB, CH), lambda i: (i, 0)),
        ],
        out_shape=[
            jax.ShapeDtypeStruct((N, CH), jnp.float32),
            jax.ShapeDtypeStruct((N, CH), jnp.float32),
        ],
    )(p0, p1, s, b, k1, k2)


def _head_body(p0_ref, p1_ref, s_ref, b_ref, gid_ref, d1w_ref, d1b_ref,
               d2w_ref, d2b_ref, out_ref, pooled_ref, cnt_ref):
    i = pl.program_id(0)

    @pl.when(i == 0)
    def _init():
        pooled_ref[...] = jnp.zeros((NG, CH), jnp.float32)
        cnt_ref[...] = jnp.zeros((NG, CH), jnp.float32)

    out2 = _elu(_elu(p0_ref[...] + p1_ref[...] + s_ref[...] + b_ref[...]))
    gids = gid_ref[0, 0, :]
    onehot = (gids[None, :] == lax.broadcasted_iota(jnp.int32, (NG, RB), 0)
              ).astype(jnp.float32)
    pooled_ref[...] += jnp.dot(onehot, out2,
                               preferred_element_type=jnp.float32)
    cnt_ref[...] += jnp.dot(onehot, jnp.ones((RB, CH), jnp.float32),
                            preferred_element_type=jnp.float32)

    @pl.when(i == NRB - 1)
    def _finish():
        pooled = pooled_ref[...] / jnp.maximum(cnt_ref[...], 1.0)
        d1 = jnp.maximum(
            jnp.dot(pooled, d1w_ref[...], preferred_element_type=jnp.float32)
            + d1b_ref[...], 0.0)
        logits = jnp.dot(d1, d2w_ref[...],
                         preferred_element_type=jnp.float32) + d2b_ref[...]
        z = logits - jnp.max(logits, axis=-1, keepdims=True)
        ez = jnp.exp(z)
        out_ref[...] = ez / jnp.sum(ez, axis=-1, keepdims=True)


def _head(p0, p1, s, b, gids3, d1w, d1b, d2w, d2b):
    return pl.pallas_call(
        _head_body,
        grid=(NRB,),
        in_specs=[
            pl.BlockSpec((RB, CH), lambda i: (i, 0)),
            pl.BlockSpec((RB, CH), lambda i: (i, 0)),
            pl.BlockSpec((RB, CH), lambda i: (i, 0)),
            pl.BlockSpec((1, CH), lambda i: (0, 0)),
            pl.BlockSpec((1, 1, RB), lambda i: (i, 0, 0)),
            pl.BlockSpec((CH, CH), lambda i: (0, 0)),
            pl.BlockSpec((1, CH), lambda i: (0, 0)),
            pl.BlockSpec((CH, NOUT), lambda i: (0, 0)),
            pl.BlockSpec((1, NOUT), lambda i: (0, 0)),
        ],
        out_specs=pl.BlockSpec((NG, NOUT), lambda i: (0, 0)),
        out_shape=jax.ShapeDtypeStruct((NG, NOUT), jnp.float32),
        scratch_shapes=[
            pltpu.VMEM((NG, CH), jnp.float32),
            pltpu.VMEM((NG, CH), jnp.float32),
        ],
    )(p0, p1, s, b, gids3, d1w, d1b, d2w, d2b)


def kernel(x, edge_weight, conv1_k1, conv1_k2, conv1_b, conv2_k1, conv2_k2,
           conv2_b, dense1_w, dense1_b, dense2_w, dense2_b, edge_index,
           graph_ids):
    pk = edge_index[0] * 16384 + edge_index[1]
    pk3 = jnp.pad(pk, (0, EPAD - E)).reshape(NTILES, NCHUNK, CHUNK)
    w3 = jnp.pad(edge_weight, (0, EPAD - E)).reshape(NTILES, NCHUNK, CHUNK)

    h1, s1 = _mm2(x, conv1_k1, conv1_k2)
    p1 = _sc_propagate(h1, pk3, w3)[:, :N]
    h2, s2 = _combine_mm2(p1[0], p1[1], s1, conv1_b.reshape(1, CH),
                          conv2_k1, conv2_k2)
    p2 = _sc_propagate(h2, pk3, w3)[:, :N]
    gids3 = graph_ids.reshape(NRB, 1, RB)
    return _head(p2[0], p2[1], s2, conv2_b.reshape(1, CH), gids3,
                 dense1_w, dense1_b.reshape(1, CH),
                 dense2_w, dense2_b.reshape(1, NOUT))


# speedup vs baseline: 1.0003x; 1.0003x over previous
import functools

import jax
import jax.numpy as jnp
from jax import lax
from jax.experimental import pallas as pl
from jax.experimental.pallas import tpu as pltpu
from jax.experimental.pallas import tpu_sc as plsc

N = 10000
E = 320000
F = 128
CH = 128
NG = 32
NOUT = 10

NCORES = 2
NSUB = 16
NTILES = NCORES * NSUB
CHUNK = 64
NCHUNK = 161
EPAD = NTILES * NCHUNK * CHUNK
NPAD = 10240
ROWS_PER_TILE = NPAD // NSUB
ZR = 16


def _sc_propagate(h, pk3, w3):
    mesh = plsc.VectorSubcoreMesh(core_axis_name="c", subcore_axis_name="s")

    @functools.partial(
        pl.kernel,
        out_type=jax.ShapeDtypeStruct((NCORES, NPAD, CH), jnp.float32),
        mesh=mesh,
        scratch_types=[
            pltpu.VMEM((NCHUNK, CHUNK), jnp.int32),
            pltpu.VMEM((CHUNK, CH), jnp.float32),
            pltpu.VMEM((CHUNK, CH), jnp.float32),
            pltpu.VMEM((CHUNK,), jnp.float32),
            pltpu.VMEM((CHUNK,), jnp.float32),
            pltpu.VMEM((CHUNK,), jnp.int32),
            pltpu.VMEM((CHUNK,), jnp.int32),
            pltpu.VMEM((CHUNK,), jnp.int32),
            pltpu.VMEM((ZR, CH), jnp.float32),
            pltpu.VMEM_SHARED((NPAD, CH), jnp.float32),
            pltpu.SemaphoreType.DMA,
            pltpu.SemaphoreType.DMA,
            pltpu.SemaphoreType.DMA,
            pltpu.SemaphoreType.DMA,
        ],
    )
    def prop(h_hbm, pk_hbm, w_hbm, out_hbm,
             mpack, rows_a, rows_b, w_a, w_b, sidx_a, sidx_b, didx,
             zbuf, acc, sem_a, sem_b, sem_wa, sem_wb):
        cid = lax.axis_index("c")
        sid = lax.axis_index("s")
        tile = cid * NSUB + sid

        pltpu.sync_copy(pk_hbm.at[tile], mpack)

        @pl.loop(0, ZR)
        def _zero(r):
            for j in range(CH // 16):
                zbuf[r, pl.ds(j * 16, 16)] = jnp.zeros((16,), jnp.float32)

        @pl.loop(0, ROWS_PER_TILE // ZR)
        def _zcopy(p_i):
            pltpu.sync_copy(
                zbuf, acc.at[pl.ds(sid * ROWS_PER_TILE + p_i * ZR, ZR)])
        plsc.subcore_barrier()

        def wait_rows(buf, sem, wbuf, sem_w):
            pltpu.make_async_copy(h_hbm.at[pl.ds(0, CHUNK)], buf, sem).wait()
            pltpu.make_async_copy(w_hbm.at[0, 0], wbuf, sem_w).wait()

        def scale_scatter(buf, wbuf, g):
            @pl.loop(0, CHUNK // 16)
            def _scale(gg):
                wvec = wbuf[pl.ds(gg * 16, 16)]
                for t in range(16):
                    e = gg * 16 + t
                    wv = jnp.full((16,), wvec[t], dtype=jnp.float32)
                    for j in range(CH // 16):
                        sl = pl.ds(j * 16, 16)
                        buf[e, sl] = buf[e, sl] * wv

            for gg in range(CHUNK // 16):
                sl = pl.ds(gg * 16, 16)
                didx[sl] = mpack[g, sl] & 16383
            pltpu.sync_copy(buf, acc.at[didx], add=True)

        def start_gather(buf, sem, wbuf, sem_w, sidx, g):
            for gg in range(CHUNK // 16):
                sl = pl.ds(gg * 16, 16)
                sidx[sl] = lax.shift_right_logical(mpack[g, sl], 14)
            pltpu.async_copy(h_hbm.at[sidx], buf, sem)
            pltpu.async_copy(w_hbm.at[tile, g], wbuf, sem_w)

        start_gather(rows_a, sem_a, w_a, sem_wa, sidx_a, 0)

        @pl.loop(0, (NCHUNK - 1) // 2)
        def _pair(p):
            g0 = 2 * p
            wait_rows(rows_a, sem_a, w_a, sem_wa)
            start_gather(rows_b, sem_b, w_b, sem_wb, sidx_b, g0 + 1)
            scale_scatter(rows_a, w_a, g0)
            wait_rows(rows_b, sem_b, w_b, sem_wb)
            start_gather(rows_a, sem_a, w_a, sem_wa, sidx_a, g0 + 2)
            scale_scatter(rows_b, w_b, g0 + 1)

        wait_rows(rows_a, sem_a, w_a, sem_wa)
        scale_scatter(rows_a, w_a, NCHUNK - 1)

        plsc.subcore_barrier()
        pltpu.sync_copy(
            acc.at[pl.ds(sid * ROWS_PER_TILE, ROWS_PER_TILE)],
            out_hbm.at[cid, pl.ds(sid * ROWS_PER_TILE, ROWS_PER_TILE)])

    return prop(h, pk3, w3)


RB = 1000
NRB = N // RB


def _elu(v):
    return jnp.where(v > 0, v, jnp.exp(v) - 1.0)


def _mm2_body(x_ref, k1_ref, k2_ref, h_ref, s_ref):
    xb = x_ref[...]
    h_ref[...] = jnp.dot(xb, k1_ref[...], preferred_element_type=jnp.float32)
    s_ref[...] = jnp.dot(xb, k2_ref[...], preferred_element_type=jnp.float32)


def _mm2(x, k1, k2):
    return pl.pallas_call(
        _mm2_body,
        grid=(NRB,),
        in_specs=[
            pl.BlockSpec((RB, F), lambda i: (i, 0)),
            pl.BlockSpec((F, CH), lambda i: (0, 0)),
            pl.BlockSpec((F, CH), lambda i: (0, 0)),
        ],
        out_specs=[
            pl.BlockSpec((RB, CH), lambda i: (i, 0)),
            pl.BlockSpec((RB, CH), lambda i: (i, 0)),
        ],
        out_shape=[
            jax.ShapeDtypeStruct((N, CH), jnp.float32),
            jax.ShapeDtypeStruct((N, CH), jnp.float32),
        ],
    )(x, k1, k2)


def _combine_mm2_body(p0_ref, p1_ref, s_ref, b_ref, k1_ref, k2_ref,
                      h_ref, s2_ref):
    out = _elu(_elu(p0_ref[...] + p1_ref[...] + s_ref[...] + b_ref[...]))
    h_ref[...] = jnp.dot(out, k1_ref[...], preferred_element_type=jnp.float32)
    s2_ref[...] = jnp.dot(out, k2_ref[...], preferred_element_type=jnp.float32)


def _combine_mm2(p0, p1, s, b, k1, k2):
    return pl.pallas_call(
        _combine_mm2_body,
        grid=(NRB,),
        in_specs=[
            pl.BlockSpec((RB, CH), lambda i: (i, 0)),
            pl.BlockSpec((RB, CH), lambda i: (i, 0)),
            pl.BlockSpec((RB, CH), lambda i: (i, 0)),
            pl.BlockSpec((1, CH), lambda i: (0, 0)),
            pl.BlockSpec((CH, CH), lambda i: (0, 0)),
            pl.BlockSpec((CH, CH), lambda i: (0, 0)),
        ],
        out_specs=[
            pl.BlockSpec((RB, CH), lambda i: (i, 0)),
            pl.BlockSpec((RB, CH), lambda i: (i, 0)),
        ],
        out_shape=[
            jax.ShapeDtypeStruct((N, CH), jnp.float32),
            jax.ShapeDtypeStruct((N, CH), jnp.float32),
        ],
    )(p0, p1, s, b, k1, k2)


def _head_body(p0_ref, p1_ref, s_ref, b_ref, gid_ref, d1w_ref, d1b_ref,
               d2w_ref, d2b_ref, out_ref, pooled_ref, cnt_ref):
    i = pl.program_id(0)

    @pl.when(i == 0)
    def _init():
        pooled_ref[...] = jnp.zeros((NG, CH), jnp.float32)
        cnt_ref[...] = jnp.zeros((NG, CH), jnp.float32)

    out2 = _elu(_elu(p0_ref[...] + p1_ref[...] + s_ref[...] + b_ref[...]))
    gids = gid_ref[0, 0, :]
    onehot = (gids[None, :] == lax.broadcasted_iota(jnp.int32, (NG, RB), 0)
              ).astype(jnp.float32)
    pooled_ref[...] += jnp.dot(onehot, out2,
                               preferred_element_type=jnp.float32)
    cnt_ref[...] += jnp.dot(onehot, jnp.ones((RB, CH), jnp.float32),
                            preferred_element_type=jnp.float32)

    @pl.when(i == NRB - 1)
    def _finish():
        pooled = pooled_ref[...] / jnp.maximum(cnt_ref[...], 1.0)
        d1 = jnp.maximum(
            jnp.dot(pooled, d1w_ref[...], preferred_element_type=jnp.float32)
            + d1b_ref[...], 0.0)
        logits = jnp.dot(d1, d2w_ref[...],
                         preferred_element_type=jnp.float32) + d2b_ref[...]
        z = logits - jnp.max(logits, axis=-1, keepdims=True)
        ez = jnp.exp(z)
        out_ref[...] = ez / jnp.sum(ez, axis=-1, keepdims=True)


def _head(p0, p1, s, b, gids3, d1w, d1b, d2w, d2b):
    return pl.pallas_call(
        _head_body,
        grid=(NRB,),
        in_specs=[
            pl.BlockSpec((RB, CH), lambda i: (i, 0)),
            pl.BlockSpec((RB, CH), lambda i: (i, 0)),
            pl.BlockSpec((RB, CH), lambda i: (i, 0)),
            pl.BlockSpec((1, CH), lambda i: (0, 0)),
            pl.BlockSpec((1, 1, RB), lambda i: (i, 0, 0)),
            pl.BlockSpec((CH, CH), lambda i: (0, 0)),
            pl.BlockSpec((1, CH), lambda i: (0, 0)),
            pl.BlockSpec((CH, NOUT), lambda i: (0, 0)),
            pl.BlockSpec((1, NOUT), lambda i: (0, 0)),
        ],
        out_specs=pl.BlockSpec((NG, NOUT), lambda i: (0, 0)),
        out_shape=jax.ShapeDtypeStruct((NG, NOUT), jnp.float32),
        scratch_shapes=[
            pltpu.VMEM((NG, CH), jnp.float32),
            pltpu.VMEM((NG, CH), jnp.float32),
        ],
    )(p0, p1, s, b, gids3, d1w, d1b, d2w, d2b)


def kernel(x, edge_weight, conv1_k1, conv1_k2, conv1_b, conv2_k1, conv2_k2,
           conv2_b, dense1_w, dense1_b, dense2_w, dense2_b, edge_index,
           graph_ids):
    pk = edge_index[0] * 16384 + edge_index[1]
    pad_dst = N + (jnp.arange(EPAD - E, dtype=jnp.int32) % (NPAD - N))
    pk3 = jnp.concatenate([pk, pad_dst]).reshape(NTILES, NCHUNK, CHUNK)
    w3 = jnp.pad(edge_weight, (0, EPAD - E)).reshape(NTILES, NCHUNK, CHUNK)

    h1, s1 = _mm2(x, conv1_k1, conv1_k2)
    p1 = _sc_propagate(h1, pk3, w3)[:, :N]
    h2, s2 = _combine_mm2(p1[0], p1[1], s1, conv1_b.reshape(1, CH),
                          conv2_k1, conv2_k2)
    p2 = _sc_propagate(h2, pk3, w3)[:, :N]
    gids3 = graph_ids.reshape(NRB, 1, RB)
    return _head(p2[0], p2[1], s2, conv2_b.reshape(1, CH), gids3,
                 dense1_w, dense1_b.reshape(1, CH),
                 dense2_w, dense2_b.reshape(1, NOUT))


# speedup vs baseline: 3.3771x; 3.3760x over previous
import functools

import jax
import jax.numpy as jnp
from jax import lax
from jax.experimental import pallas as pl
from jax.experimental.pallas import tpu as pltpu
from jax.experimental.pallas import tpu_sc as plsc

N = 10000
E = 320000
F = 128
CH = 128
NG = 32
NOUT = 10

NCORES = 2
NSUB = 16
NTILES = NCORES * NSUB
CHUNK = 96
NCHUNK = 105
EPAD = NTILES * NCHUNK * CHUNK
NPAD = 10240
ROWS_PER_TILE = NPAD // NSUB
ZR = 16


def _sc_propagate(h, pk3, w3):
    mesh = plsc.VectorSubcoreMesh(core_axis_name="c", subcore_axis_name="s")

    @functools.partial(
        pl.kernel,
        out_type=jax.ShapeDtypeStruct((NCORES, NPAD, CH), jnp.float32),
        mesh=mesh,
        scratch_types=[
            pltpu.VMEM((NCHUNK, CHUNK), jnp.int32),
            pltpu.VMEM((CHUNK, CH), jnp.float32),
            pltpu.VMEM((CHUNK, CH), jnp.float32),
            pltpu.VMEM((CHUNK,), jnp.float32),
            pltpu.VMEM((CHUNK,), jnp.float32),
            pltpu.VMEM((CHUNK,), jnp.int32),
            pltpu.VMEM((CHUNK,), jnp.int32),
            pltpu.VMEM((CHUNK,), jnp.int32),
            pltpu.VMEM((ZR, CH), jnp.float32),
            pltpu.VMEM_SHARED((NPAD, CH), jnp.float32),
            pltpu.SemaphoreType.DMA,
            pltpu.SemaphoreType.DMA,
            pltpu.SemaphoreType.DMA,
            pltpu.SemaphoreType.DMA,
        ],
    )
    def prop(h_hbm, pk_hbm, w_hbm, out_hbm,
             mpack, rows_a, rows_b, w_a, w_b, sidx_a, sidx_b, didx,
             zbuf, acc, sem_a, sem_b, sem_wa, sem_wb):
        cid = lax.axis_index("c")
        sid = lax.axis_index("s")
        tile = cid * NSUB + sid

        pltpu.sync_copy(pk_hbm.at[tile], mpack)

        @pl.loop(0, ZR)
        def _zero(r):
            for j in range(CH // 16):
                zbuf[r, pl.ds(j * 16, 16)] = jnp.zeros((16,), jnp.float32)

        @pl.loop(0, ROWS_PER_TILE // ZR)
        def _zcopy(p_i):
            pltpu.sync_copy(
                zbuf, acc.at[pl.ds(sid * ROWS_PER_TILE + p_i * ZR, ZR)])
        plsc.subcore_barrier()

        def wait_rows(buf, sem, wbuf, sem_w):
            pltpu.make_async_copy(h_hbm.at[pl.ds(0, CHUNK)], buf, sem).wait()
            pltpu.make_async_copy(w_hbm.at[0, 0], wbuf, sem_w).wait()

        def scale_scatter(buf, wbuf, g):
            @pl.loop(0, CHUNK // 16)
            def _scale(gg):
                wvec = wbuf[pl.ds(gg * 16, 16)]
                for t in range(16):
                    e = gg * 16 + t
                    wv = jnp.full((16,), wvec[t], dtype=jnp.float32)
                    for j in range(CH // 16):
                        sl = pl.ds(j * 16, 16)
                        buf[e, sl] = buf[e, sl] * wv

            for gg in range(CHUNK // 16):
                sl = pl.ds(gg * 16, 16)
                didx[sl] = mpack[g, sl] & 16383
            pltpu.sync_copy(buf, acc.at[didx], add=True)

        def start_gather(buf, sem, wbuf, sem_w, sidx, g):
            for gg in range(CHUNK // 16):
                sl = pl.ds(gg * 16, 16)
                sidx[sl] = lax.shift_right_logical(mpack[g, sl], 14)
            pltpu.async_copy(h_hbm.at[sidx], buf, sem)
            pltpu.async_copy(w_hbm.at[tile, g], wbuf, sem_w)

        start_gather(rows_a, sem_a, w_a, sem_wa, sidx_a, 0)

        @pl.loop(0, (NCHUNK - 1) // 2)
        def _pair(p):
            g0 = 2 * p
            wait_rows(rows_a, sem_a, w_a, sem_wa)
            start_gather(rows_b, sem_b, w_b, sem_wb, sidx_b, g0 + 1)
            scale_scatter(rows_a, w_a, g0)
            wait_rows(rows_b, sem_b, w_b, sem_wb)
            start_gather(rows_a, sem_a, w_a, sem_wa, sidx_a, g0 + 2)
            scale_scatter(rows_b, w_b, g0 + 1)

        wait_rows(rows_a, sem_a, w_a, sem_wa)
        scale_scatter(rows_a, w_a, NCHUNK - 1)

        plsc.subcore_barrier()
        pltpu.sync_copy(
            acc.at[pl.ds(sid * ROWS_PER_TILE, ROWS_PER_TILE)],
            out_hbm.at[cid, pl.ds(sid * ROWS_PER_TILE, ROWS_PER_TILE)])

    return prop(h, pk3, w3)


RB = 1000
NRB = N // RB


def _elu(v):
    return jnp.where(v > 0, v, jnp.exp(v) - 1.0)


def _mm2_body(x_ref, k1_ref, k2_ref, h_ref, s_ref):
    xb = x_ref[...]
    h_ref[...] = jnp.dot(xb, k1_ref[...], preferred_element_type=jnp.float32)
    s_ref[...] = jnp.dot(xb, k2_ref[...], preferred_element_type=jnp.float32)


def _mm2(x, k1, k2):
    return pl.pallas_call(
        _mm2_body,
        grid=(NRB,),
        in_specs=[
            pl.BlockSpec((RB, F), lambda i: (i, 0)),
            pl.BlockSpec((F, CH), lambda i: (0, 0)),
            pl.BlockSpec((F, CH), lambda i: (0, 0)),
        ],
        out_specs=[
            pl.BlockSpec((RB, CH), lambda i: (i, 0)),
            pl.BlockSpec((RB, CH), lambda i: (i, 0)),
        ],
        out_shape=[
            jax.ShapeDtypeStruct((N, CH), jnp.float32),
            jax.ShapeDtypeStruct((N, CH), jnp.float32),
        ],
    )(x, k1, k2)


def _combine_mm2_body(p0_ref, p1_ref, s_ref, b_ref, k1_ref, k2_ref,
                      h_ref, s2_ref):
    out = _elu(_elu(p0_ref[...] + p1_ref[...] + s_ref[...] + b_ref[...]))
    h_ref[...] = jnp.dot(out, k1_ref[...], preferred_element_type=jnp.float32)
    s2_ref[...] = jnp.dot(out, k2_ref[...], preferred_element_type=jnp.float32)


def _combine_mm2(p0, p1, s, b, k1, k2):
    return pl.pallas_call(
        _combine_mm2_body,
        grid=(NRB,),
        in_specs=[
            pl.BlockSpec((RB, CH), lambda i: (i, 0)),
            pl.BlockSpec((RB, CH), lambda i: (i, 0)),
            pl.BlockSpec((RB, CH), lambda i: (i, 0)),
            pl.BlockSpec((1, CH), lambda i: (0, 0)),
            pl.BlockSpec((CH, CH), lambda i: (0, 0)),
            pl.BlockSpec((CH, CH), lambda i: (0, 0)),
        ],
        out_specs=[
            pl.BlockSpec((RB, CH), lambda i: (i, 0)),
            pl.BlockSpec((RB, CH), lambda i: (i, 0)),
        ],
        out_shape=[
            jax.ShapeDtypeStruct((N, CH), jnp.float32),
            jax.ShapeDtypeStruct((N, CH), jnp.float32),
        ],
    )(p0, p1, s, b, k1, k2)


def _head_body(p0_ref, p1_ref, s_ref, b_ref, gid_ref, d1w_ref, d1b_ref,
               d2w_ref, d2b_ref, out_ref, pooled_ref, cnt_ref):
    i = pl.program_id(0)

    @pl.when(i == 0)
    def _init():
        pooled_ref[...] = jnp.zeros((NG, CH), jnp.float32)
        cnt_ref[...] = jnp.zeros((NG, CH), jnp.float32)

    out2 = _elu(_elu(p0_ref[...] + p1_ref[...] + s_ref[...] + b_ref[...]))
    gids = gid_ref[0, 0, :]
    onehot = (gids[None, :] == lax.broadcasted_iota(jnp.int32, (NG, RB), 0)
              ).astype(jnp.float32)
    pooled_ref[...] += jnp.dot(onehot, out2,
                               preferred_element_type=jnp.float32)
    cnt_ref[...] += jnp.dot(onehot, jnp.ones((RB, CH), jnp.float32),
                            preferred_element_type=jnp.float32)

    @pl.when(i == NRB - 1)
    def _finish():
        pooled = pooled_ref[...] / jnp.maximum(cnt_ref[...], 1.0)
        d1 = jnp.maximum(
            jnp.dot(pooled, d1w_ref[...], preferred_element_type=jnp.float32)
            + d1b_ref[...], 0.0)
        logits = jnp.dot(d1, d2w_ref[...],
                         preferred_element_type=jnp.float32) + d2b_ref[...]
        z = logits - jnp.max(logits, axis=-1, keepdims=True)
        ez = jnp.exp(z)
        out_ref[...] = ez / jnp.sum(ez, axis=-1, keepdims=True)


def _head(p0, p1, s, b, gids3, d1w, d1b, d2w, d2b):
    return pl.pallas_call(
        _head_body,
        grid=(NRB,),
        in_specs=[
            pl.BlockSpec((RB, CH), lambda i: (i, 0)),
            pl.BlockSpec((RB, CH), lambda i: (i, 0)),
            pl.BlockSpec((RB, CH), lambda i: (i, 0)),
            pl.BlockSpec((1, CH), lambda i: (0, 0)),
            pl.BlockSpec((1, 1, RB), lambda i: (i, 0, 0)),
            pl.BlockSpec((CH, CH), lambda i: (0, 0)),
            pl.BlockSpec((1, CH), lambda i: (0, 0)),
            pl.BlockSpec((CH, NOUT), lambda i: (0, 0)),
            pl.BlockSpec((1, NOUT), lambda i: (0, 0)),
        ],
        out_specs=pl.BlockSpec((NG, NOUT), lambda i: (0, 0)),
        out_shape=jax.ShapeDtypeStruct((NG, NOUT), jnp.float32),
        scratch_shapes=[
            pltpu.VMEM((NG, CH), jnp.float32),
            pltpu.VMEM((NG, CH), jnp.float32),
        ],
    )(p0, p1, s, b, gids3, d1w, d1b, d2w, d2b)


def kernel(x, edge_weight, conv1_k1, conv1_k2, conv1_b, conv2_k1, conv2_k2,
           conv2_b, dense1_w, dense1_b, dense2_w, dense2_b, edge_index,
           graph_ids):
    pk = edge_index[0] * 16384 + edge_index[1]
    npd = EPAD - E
    pad_src = jnp.arange(npd, dtype=jnp.int32) % N
    pad_dst = N + (jnp.arange(npd, dtype=jnp.int32) % (NPAD - N))
    pk3 = jnp.concatenate([pk, pad_src * 16384 + pad_dst]).reshape(
        NTILES, NCHUNK, CHUNK)
    w3 = jnp.pad(edge_weight, (0, EPAD - E)).reshape(NTILES, NCHUNK, CHUNK)

    h1, s1 = _mm2(x, conv1_k1, conv1_k2)
    p1 = _sc_propagate(h1, pk3, w3)[:, :N]
    h2, s2 = _combine_mm2(p1[0], p1[1], s1, conv1_b.reshape(1, CH),
                          conv2_k1, conv2_k2)
    p2 = _sc_propagate(h2, pk3, w3)[:, :N]
    gids3 = graph_ids.reshape(NRB, 1, RB)
    return _head(p2[0], p2[1], s2, conv2_b.reshape(1, CH), gids3,
                 dense1_w, dense1_b.reshape(1, CH),
                 dense2_w, dense2_b.reshape(1, NOUT))


# speedup vs baseline: 3.5025x; 1.0371x over previous
import functools

import jax
import jax.numpy as jnp
from jax import lax
from jax.experimental import pallas as pl
from jax.experimental.pallas import tpu as pltpu
from jax.experimental.pallas import tpu_sc as plsc

N = 10000
E = 320000
F = 128
CH = 128
NG = 32
NOUT = 10

NCORES = 2
NSUB = 16
NTILES = NCORES * NSUB
CHUNK = 112
NCHUNK = 91
EPAD = NTILES * NCHUNK * CHUNK
NPAD = 10240
ROWS_PER_TILE = NPAD // NSUB
ZR = 16


def _sc_propagate(h, pk3, w3):
    mesh = plsc.VectorSubcoreMesh(core_axis_name="c", subcore_axis_name="s")

    @functools.partial(
        pl.kernel,
        out_type=jax.ShapeDtypeStruct((NCORES, NPAD, CH), jnp.float32),
        mesh=mesh,
        scratch_types=[
            pltpu.VMEM((NCHUNK, CHUNK), jnp.int32),
            pltpu.VMEM((CHUNK, CH), jnp.float32),
            pltpu.VMEM((CHUNK, CH), jnp.float32),
            pltpu.VMEM((CHUNK,), jnp.float32),
            pltpu.VMEM((CHUNK,), jnp.float32),
            pltpu.VMEM((CHUNK,), jnp.int32),
            pltpu.VMEM((CHUNK,), jnp.int32),
            pltpu.VMEM((CHUNK,), jnp.int32),
            pltpu.VMEM((ZR, CH), jnp.float32),
            pltpu.VMEM_SHARED((NPAD, CH), jnp.float32),
            pltpu.SemaphoreType.DMA,
            pltpu.SemaphoreType.DMA,
            pltpu.SemaphoreType.DMA,
            pltpu.SemaphoreType.DMA,
        ],
    )
    def prop(h_hbm, pk_hbm, w_hbm, out_hbm,
             mpack, rows_a, rows_b, w_a, w_b, sidx_a, sidx_b, didx,
             zbuf, acc, sem_a, sem_b, sem_wa, sem_wb):
        cid = lax.axis_index("c")
        sid = lax.axis_index("s")
        tile = cid * NSUB + sid

        pltpu.sync_copy(pk_hbm.at[tile], mpack)

        @pl.loop(0, ZR)
        def _zero(r):
            for j in range(CH // 16):
                zbuf[r, pl.ds(j * 16, 16)] = jnp.zeros((16,), jnp.float32)

        @pl.loop(0, ROWS_PER_TILE // ZR)
        def _zcopy(p_i):
            pltpu.sync_copy(
                zbuf, acc.at[pl.ds(sid * ROWS_PER_TILE + p_i * ZR, ZR)])
        plsc.subcore_barrier()

        def wait_rows(buf, sem, wbuf, sem_w):
            pltpu.make_async_copy(h_hbm.at[pl.ds(0, CHUNK)], buf, sem).wait()
            pltpu.make_async_copy(w_hbm.at[0, 0], wbuf, sem_w).wait()

        def scale_scatter(buf, wbuf, g):
            @pl.loop(0, CHUNK // 16)
            def _scale(gg):
                wvec = wbuf[pl.ds(gg * 16, 16)]
                for t in range(16):
                    e = gg * 16 + t
                    wv = jnp.full((16,), wvec[t], dtype=jnp.float32)
                    for j in range(CH // 16):
                        sl = pl.ds(j * 16, 16)
                        buf[e, sl] = buf[e, sl] * wv

            for gg in range(CHUNK // 16):
                sl = pl.ds(gg * 16, 16)
                didx[sl] = mpack[g, sl] & 16383
            pltpu.sync_copy(buf, acc.at[didx], add=True)

        def start_gather(buf, sem, wbuf, sem_w, sidx, g):
            for gg in range(CHUNK // 16):
                sl = pl.ds(gg * 16, 16)
                sidx[sl] = lax.shift_right_logical(mpack[g, sl], 14)
            pltpu.async_copy(h_hbm.at[sidx], buf, sem)
            pltpu.async_copy(w_hbm.at[tile, g], wbuf, sem_w)

        start_gather(rows_a, sem_a, w_a, sem_wa, sidx_a, 0)

        @pl.loop(0, (NCHUNK - 1) // 2)
        def _pair(p):
            g0 = 2 * p
            wait_rows(rows_a, sem_a, w_a, sem_wa)
            start_gather(rows_b, sem_b, w_b, sem_wb, sidx_b, g0 + 1)
            scale_scatter(rows_a, w_a, g0)
            wait_rows(rows_b, sem_b, w_b, sem_wb)
            start_gather(rows_a, sem_a, w_a, sem_wa, sidx_a, g0 + 2)
            scale_scatter(rows_b, w_b, g0 + 1)

        wait_rows(rows_a, sem_a, w_a, sem_wa)
        scale_scatter(rows_a, w_a, NCHUNK - 1)

        plsc.subcore_barrier()
        pltpu.sync_copy(
            acc.at[pl.ds(sid * ROWS_PER_TILE, ROWS_PER_TILE)],
            out_hbm.at[cid, pl.ds(sid * ROWS_PER_TILE, ROWS_PER_TILE)])

    return prop(h, pk3, w3)


RB = 1000
NRB = N // RB


def _elu(v):
    return jnp.where(v > 0, v, jnp.exp(v) - 1.0)


def _mm2_body(x_ref, k1_ref, k2_ref, h_ref, s_ref):
    xb = x_ref[...]
    h_ref[...] = jnp.dot(xb, k1_ref[...], preferred_element_type=jnp.float32)
    s_ref[...] = jnp.dot(xb, k2_ref[...], preferred_element_type=jnp.float32)


def _mm2(x, k1, k2):
    return pl.pallas_call(
        _mm2_body,
        grid=(NRB,),
        in_specs=[
            pl.BlockSpec((RB, F), lambda i: (i, 0)),
            pl.BlockSpec((F, CH), lambda i: (0, 0)),
            pl.BlockSpec((F, CH), lambda i: (0, 0)),
        ],
        out_specs=[
            pl.BlockSpec((RB, CH), lambda i: (i, 0)),
            pl.BlockSpec((RB, CH), lambda i: (i, 0)),
        ],
        out_shape=[
            jax.ShapeDtypeStruct((N, CH), jnp.float32),
            jax.ShapeDtypeStruct((N, CH), jnp.float32),
        ],
    )(x, k1, k2)


def _combine_mm2_body(p0_ref, p1_ref, s_ref, b_ref, k1_ref, k2_ref,
                      h_ref, s2_ref):
    out = _elu(_elu(p0_ref[...] + p1_ref[...] + s_ref[...] + b_ref[...]))
    h_ref[...] = jnp.dot(out, k1_ref[...], preferred_element_type=jnp.float32)
    s2_ref[...] = jnp.dot(out, k2_ref[...], preferred_element_type=jnp.float32)


def _combine_mm2(p0, p1, s, b, k1, k2):
    return pl.pallas_call(
        _combine_mm2_body,
        grid=(NRB,),
        in_specs=[
            pl.BlockSpec((RB, CH), lambda i: (i, 0)),
            pl.BlockSpec((RB, CH), lambda i: (i, 0)),
            pl.BlockSpec((RB, CH), lambda i: (i, 0)),
            pl.BlockSpec((1, CH), lambda i: (0, 0)),
            pl.BlockSpec((CH, CH), lambda i: (0, 0)),
            pl.BlockSpec((CH, CH), lambda i: (0, 0)),
        ],
        out_specs=[
            pl.BlockSpec((RB, CH), lambda i: (i, 0)),
            pl.BlockSpec((RB, CH), lambda i: (i, 0)),
        ],
        out_shape=[
            jax.ShapeDtypeStruct((N, CH), jnp.float32),
            jax.ShapeDtypeStruct((N, CH), jnp.float32),
        ],
    )(p0, p1, s, b, k1, k2)


def _head_body(p0_ref, p1_ref, s_ref, b_ref, gid_ref, d1w_ref, d1b_ref,
               d2w_ref, d2b_ref, out_ref, pooled_ref, cnt_ref):
    i = pl.program_id(0)

    @pl.when(i == 0)
    def _init():
        pooled_ref[...] = jnp.zeros((NG, CH), jnp.float32)
        cnt_ref[...] = jnp.zeros((NG, CH), jnp.float32)

    out2 = _elu(_elu(p0_ref[...] + p1_ref[...] + s_ref[...] + b_ref[...]))
    gids = gid_ref[0, 0, :]
    onehot = (gids[None, :] == lax.broadcasted_iota(jnp.int32, (NG, RB), 0)
              ).astype(jnp.float32)
    pooled_ref[...] += jnp.dot(onehot, out2,
                               preferred_element_type=jnp.float32)
    cnt_ref[...] += jnp.dot(onehot, jnp.ones((RB, CH), jnp.float32),
                            preferred_element_type=jnp.float32)

    @pl.when(i == NRB - 1)
    def _finish():
        pooled = pooled_ref[...] / jnp.maximum(cnt_ref[...], 1.0)
        d1 = jnp.maximum(
            jnp.dot(pooled, d1w_ref[...], preferred_element_type=jnp.float32)
            + d1b_ref[...], 0.0)
        logits = jnp.dot(d1, d2w_ref[...],
                         preferred_element_type=jnp.float32) + d2b_ref[...]
        z = logits - jnp.max(logits, axis=-1, keepdims=True)
        ez = jnp.exp(z)
        out_ref[...] = ez / jnp.sum(ez, axis=-1, keepdims=True)


def _head(p0, p1, s, b, gids3, d1w, d1b, d2w, d2b):
    return pl.pallas_call(
        _head_body,
        grid=(NRB,),
        in_specs=[
            pl.BlockSpec((RB, CH), lambda i: (i, 0)),
            pl.BlockSpec((RB, CH), lambda i: (i, 0)),
            pl.BlockSpec((RB, CH), lambda i: (i, 0)),
            pl.BlockSpec((1, CH), lambda i: (0, 0)),
            pl.BlockSpec((1, 1, RB), lambda i: (i, 0, 0)),
            pl.BlockSpec((CH, CH), lambda i: (0, 0)),
            pl.BlockSpec((1, CH), lambda i: (0, 0)),
            pl.BlockSpec((CH, NOUT), lambda i: (0, 0)),
            pl.BlockSpec((1, NOUT), lambda i: (0, 0)),
        ],
        out_specs=pl.BlockSpec((NG, NOUT), lambda i: (0, 0)),
        out_shape=jax.ShapeDtypeStruct((NG, NOUT), jnp.float32),
        scratch_shapes=[
            pltpu.VMEM((NG, CH), jnp.float32),
            pltpu.VMEM((NG, CH), jnp.float32),
        ],
    )(p0, p1, s, b, gids3, d1w, d1b, d2w, d2b)


def kernel(x, edge_weight, conv1_k1, conv1_k2, conv1_b, conv2_k1, conv2_k2,
           conv2_b, dense1_w, dense1_b, dense2_w, dense2_b, edge_index,
           graph_ids):
    pk = edge_index[0] * 16384 + edge_index[1]
    npd = EPAD - E
    pad_src = jnp.arange(npd, dtype=jnp.int32) % N
    pad_dst = N + (jnp.arange(npd, dtype=jnp.int32) % (NPAD - N))
    pk3 = jnp.concatenate([pk, pad_src * 16384 + pad_dst]).reshape(
        NTILES, NCHUNK, CHUNK)
    w3 = jnp.pad(edge_weight, (0, EPAD - E)).reshape(NTILES, NCHUNK, CHUNK)

    h1, s1 = _mm2(x, conv1_k1, conv1_k2)
    p1 = _sc_propagate(h1, pk3, w3)[:, :N]
    h2, s2 = _combine_mm2(p1[0], p1[1], s1, conv1_b.reshape(1, CH),
                          conv2_k1, conv2_k2)
    p2 = _sc_propagate(h2, pk3, w3)[:, :N]
    gids3 = graph_ids.reshape(NRB, 1, RB)
    return _head(p2[0], p2[1], s2, conv2_b.reshape(1, CH), gids3,
                 dense1_w, dense1_b.reshape(1, CH),
                 dense2_w, dense2_b.reshape(1, NOUT))
